# baseline (speedup 1.0000x reference)
import math

import jax
import jax.numpy as jnp
from jax import lax
from jax.experimental import pallas as pl
from jax.experimental.pallas import tpu as pltpu
from jax.experimental.pallas import tpu_sc as plsc

NC, NS, L = 2, 16, 16
NW = NC * NS

BETA1, BETA2 = 0.9, 0.999
EPS = 1e-15
LR = 1e-3
STEP = 1
_BC1 = 1.0 - BETA1 ** STEP
_BC2 = 1.0 - BETA2 ** STEP
STEP_SIZE = LR / _BC1
INV_SQRT_BC2 = 1.0 / math.sqrt(_BC2)

CHUNK = 128
TCB_T = 2048
TCB_G = 512


def _remap(p, tcb):
    h = tcb // 2
    return 2 * (h * (p // tcb) + (p % h)) + ((p // h) & 1)


def _t_in_body(*refs):
    n = len(refs) // 2
    h = refs[0].shape[1] // 2
    for x, o in zip(refs[:n], refs[n:]):
        o[:, 0:64] = jnp.transpose(x[:, 0:h])
        o[:, 64:128] = jnp.transpose(x[:, h:2 * h])


def _t_in(xs, tcb):
    N = xs[0].shape[1]
    nb = pl.cdiv(N, tcb)
    rpad = nb * (tcb // 2)
    out = pl.pallas_call(
        _t_in_body,
        grid=(nb,),
        in_specs=[pl.BlockSpec((64, tcb), lambda i: (0, i))] * len(xs),
        out_specs=[pl.BlockSpec((tcb // 2, 128), lambda i: (i, 0))] * len(xs),
        out_shape=[jax.ShapeDtypeStruct((rpad, 128), jnp.float32)] * len(xs),
    )(*xs)
    return out


def _t_out_body(*refs):
    n = len(refs) // 2
    h = refs[n].shape[1] // 2
    for x, o in zip(refs[:n], refs[n:]):
        o[:, 0:h] = jnp.transpose(x[:, 0:64])
        o[:, h:2 * h] = jnp.transpose(x[:, 64:128])


def _t_out(xs, N, tcb):
    nb = pl.cdiv(N, tcb)
    out = pl.pallas_call(
        _t_out_body,
        grid=(nb,),
        in_specs=[pl.BlockSpec((tcb // 2, 128), lambda i: (i, 0))] * len(xs),
        out_specs=[pl.BlockSpec((64, tcb), lambda i: (0, i))] * len(xs),
        out_shape=[jax.ShapeDtypeStruct((64, N), jnp.float32)] * len(xs),
    )(*xs)
    return out


def _adam_vec(p, ea, eas, g):
    ea2 = ea * BETA1 + g * (1.0 - BETA1)
    eas2 = eas * BETA2 + (g * g) * (1.0 - BETA2)
    bits = lax.bitcast_convert_type(eas2, jnp.int32)
    y = lax.bitcast_convert_type(jnp.int32(0x5F3759DF) - (bits >> 1),
                                 jnp.float32)
    xh = eas2 * 0.5
    y = y * (1.5 - xh * y * y)
    y = y * (1.5 - xh * y * y)
    y = y * (1.5 - xh * y * y)
    root = eas2 * y
    denom = root * INV_SQRT_BC2 + EPS
    p2 = p - STEP_SIZE * (ea2 / denom)
    return p2, ea2, eas2


def _stage_grad(g_hbm2, g_v, c):
    base_g = 256 * (c // 4) + 128 * (c % 2)
    c0 = 64 * ((c // 2) % 2)
    pltpu.sync_copy(g_hbm2.at[pl.ds(base_g, CHUNK), pl.ds(c0, 64)], g_v)


def _sc_body_p(p_hbm, ea_hbm, eas_hbm, g_hbm2, idx_hbm,
               idx_v, p_v, ea_v, eas_v, g_v, sem):
    wid = lax.axis_index("s") * NC + lax.axis_index("c")
    n_chunks = idx_hbm.shape[0] // NW
    pltpu.sync_copy(idx_hbm.at[pl.ds(wid * n_chunks, n_chunks)], idx_v)

    def chunk_body(j, _):
        _stage_grad(g_hbm2, g_v, wid * n_chunks + j)
        idx_row = idx_v.at[j]
        pltpu.async_copy(p_hbm.at[idx_row], p_v, sem).wait()
        pltpu.async_copy(ea_hbm.at[idx_row], ea_v, sem).wait()
        pltpu.async_copy(eas_hbm.at[idx_row], eas_v, sem).wait()

        def row_body(i, _):
            for l in range(4):
                sl = pl.ds(l * L, L)
                p2, _, _ = _adam_vec(
                    p_v[i, sl], ea_v[i, sl], eas_v[i, sl], g_v[i, sl])
                p_v[i, sl] = p2
            return 0

        lax.fori_loop(0, CHUNK, row_body, 0)
        pltpu.async_copy(p_v, p_hbm.at[idx_row], sem).wait()
        return 0

    lax.fori_loop(0, n_chunks, chunk_body, 0)


def _sc_body_m(ea_hbm, eas_hbm, g_hbm2, idx_hbm,
               idx_v, ea_v, eas_v, g_v, sem):
    wid = lax.axis_index("s") * NC + lax.axis_index("c")
    n_chunks = idx_hbm.shape[0] // NW
    pltpu.sync_copy(idx_hbm.at[pl.ds(wid * n_chunks, n_chunks)], idx_v)

    def chunk_body(j, _):
        _stage_grad(g_hbm2, g_v, wid * n_chunks + j)
        idx_row = idx_v.at[j]
        pltpu.async_copy(ea_hbm.at[idx_row], ea_v, sem).wait()
        pltpu.async_copy(eas_hbm.at[idx_row], eas_v, sem).wait()

        def row_body(i, _):
            for l in range(4):
                sl = pl.ds(l * L, L)
                g = g_v[i, sl]
                ea_v[i, sl] = ea_v[i, sl] * BETA1 + g * (1.0 - BETA1)
                eas_v[i, sl] = eas_v[i, sl] * BETA2 + (g * g) * (1.0 - BETA2)
            return 0

        lax.fori_loop(0, CHUNK, row_body, 0)
        pltpu.async_copy(ea_v, ea_hbm.at[idx_row], sem).wait()
        pltpu.async_copy(eas_v, eas_hbm.at[idx_row], sem).wait()
        return 0

    lax.fori_loop(0, n_chunks, chunk_body, 0)


def kernel(param, grad, exp_avg, exp_avg_sq, index):
    M, D = param.shape
    B = index.shape[0]
    assert B % (NW * 4 * CHUNK) == 0 and D == 64

    idxr = _remap(index.astype(jnp.int32), TCB_T)
    idx2d = idxr.reshape(B // CHUNK, CHUNK)

    p2, ea2, eas2 = _t_in([param.T, exp_avg.T, exp_avg_sq.T], TCB_T)
    (g2,) = _t_in([grad.T], TCB_G)

    mpad = 2 * p2.shape[0]
    p_ref = jax.new_ref(p2.reshape(mpad, D))
    ea_ref = jax.new_ref(ea2.reshape(mpad, D))
    eas_ref = jax.new_ref(eas2.reshape(mpad, D))

    mesh = plsc.VectorSubcoreMesh(
        core_axis_name="c", subcore_axis_name="s",
        num_cores=NC, num_subcores=NS)
    n_chunks = (B // CHUNK) // NW
    cp = pltpu.CompilerParams(use_tc_tiling_on_sc=False)
    _vm = lambda: pltpu.VMEM((CHUNK, D), jnp.float32)
    sc_p = pl.kernel(
        _sc_body_p, out_type=(), mesh=mesh, compiler_params=cp,
        scratch_types=[
            pltpu.VMEM((n_chunks, CHUNK), jnp.int32),
            _vm(), _vm(), _vm(), _vm(),
            pltpu.SemaphoreType.DMA,
        ],
    )
    sc_m = pl.kernel(
        _sc_body_m, out_type=(), mesh=mesh, compiler_params=cp,
        scratch_types=[
            pltpu.VMEM((n_chunks, CHUNK), jnp.int32),
            _vm(), _vm(), _vm(),
            pltpu.SemaphoreType.DMA,
        ],
    )
    sc_p(p_ref, ea_ref, eas_ref, g2, idx2d)
    (po,) = _t_out([p_ref[...].reshape(mpad // 2, 2 * D)], M, TCB_T)
    sc_m(ea_ref, eas_ref, g2, idx2d)
    eao, easo = _t_out(
        [ea_ref[...].reshape(mpad // 2, 2 * D),
         eas_ref[...].reshape(mpad // 2, 2 * D)], M, TCB_T)
    return po.T, eao.T, easo.T


# speedup vs baseline: 1.1274x; 1.1274x over previous
import math

import jax
import jax.numpy as jnp
from jax import lax
from jax.experimental import pallas as pl
from jax.experimental.pallas import tpu as pltpu
from jax.experimental.pallas import tpu_sc as plsc

NC, NS, L = 2, 16, 16
NW = NC * NS

BETA1, BETA2 = 0.9, 0.999
EPS = 1e-15
LR = 1e-3
STEP = 1
_BC1 = 1.0 - BETA1 ** STEP
_BC2 = 1.0 - BETA2 ** STEP
STEP_SIZE = LR / _BC1
INV_SQRT_BC2 = 1.0 / math.sqrt(_BC2)

CHUNK = 128
TCB_T = 2048
TCB_G = 512


def _remap(p, tcb):
    h = tcb // 2
    return 2 * (h * (p // tcb) + (p % h)) + ((p // h) & 1)


def _t_in_body(*refs):
    n = len(refs) // 2
    h = refs[0].shape[1] // 2
    for x, o in zip(refs[:n], refs[n:]):
        o[:, 0:64] = jnp.transpose(x[:, 0:h])
        o[:, 64:128] = jnp.transpose(x[:, h:2 * h])


def _t_in(xs, tcb):
    N = xs[0].shape[1]
    nb = pl.cdiv(N, tcb)
    rpad = nb * (tcb // 2)
    out = pl.pallas_call(
        _t_in_body,
        grid=(nb,),
        in_specs=[pl.BlockSpec((64, tcb), lambda i: (0, i))] * len(xs),
        out_specs=[pl.BlockSpec((tcb // 2, 128), lambda i: (i, 0))] * len(xs),
        out_shape=[jax.ShapeDtypeStruct((rpad, 128), jnp.float32)] * len(xs),
    )(*xs)
    return out


def _t_out_body(*refs):
    n = len(refs) // 2
    h = refs[n].shape[1] // 2
    for x, o in zip(refs[:n], refs[n:]):
        o[:, 0:h] = jnp.transpose(x[:, 0:64])
        o[:, h:2 * h] = jnp.transpose(x[:, 64:128])


def _t_out(xs, N, tcb):
    nb = pl.cdiv(N, tcb)
    out = pl.pallas_call(
        _t_out_body,
        grid=(nb,),
        in_specs=[pl.BlockSpec((tcb // 2, 128), lambda i: (i, 0))] * len(xs),
        out_specs=[pl.BlockSpec((64, tcb), lambda i: (0, i))] * len(xs),
        out_shape=[jax.ShapeDtypeStruct((64, N), jnp.float32)] * len(xs),
    )(*xs)
    return out


def _adam_vec(p, ea, eas, g):
    ea2 = ea * BETA1 + g * (1.0 - BETA1)
    eas2 = eas * BETA2 + (g * g) * (1.0 - BETA2)
    bits = lax.bitcast_convert_type(eas2, jnp.int32)
    y = lax.bitcast_convert_type(jnp.int32(0x5F3759DF) - (bits >> 1),
                                 jnp.float32)
    xh = eas2 * 0.5
    y = y * (1.5 - xh * y * y)
    y = y * (1.5 - xh * y * y)
    y = y * (1.5 - xh * y * y)
    root = eas2 * y
    denom = root * INV_SQRT_BC2 + EPS
    p2 = p - STEP_SIZE * (ea2 / denom)
    return p2, ea2, eas2


def _stage_grad(g_hbm2, g_v, c):
    base_g = 256 * (c // 4) + 128 * (c % 2)
    c0 = 64 * ((c // 2) % 2)
    pltpu.sync_copy(g_hbm2.at[pl.ds(base_g, CHUNK), pl.ds(c0, 64)], g_v)


def _sc_body(p_hbm, ea_hbm, eas_hbm, g_hbm2, idx_hbm,
             idx_v, p_v, ea_v, eas_v, g_v, sem):
    wid = lax.axis_index("s") * NC + lax.axis_index("c")
    n_chunks = idx_hbm.shape[0] // NW
    pltpu.sync_copy(idx_hbm.at[pl.ds(wid * n_chunks, n_chunks)], idx_v)

    def chunk_body(j, _):
        _stage_grad(g_hbm2, g_v, wid * n_chunks + j)
        idx_row = idx_v.at[j]
        d1 = pltpu.async_copy(p_hbm.at[idx_row], p_v, sem)
        d2 = pltpu.async_copy(ea_hbm.at[idx_row], ea_v, sem)
        d3 = pltpu.async_copy(eas_hbm.at[idx_row], eas_v, sem)
        d1.wait()
        d2.wait()
        d3.wait()

        def row_body(i, _):
            for l in range(4):
                sl = pl.ds(l * L, L)
                p2, ea2, eas2 = _adam_vec(
                    p_v[i, sl], ea_v[i, sl], eas_v[i, sl], g_v[i, sl])
                p_v[i, sl] = p2
                ea_v[i, sl] = ea2
                eas_v[i, sl] = eas2
            return 0

        lax.fori_loop(0, CHUNK, row_body, 0)

        s1 = pltpu.async_copy(p_v, p_hbm.at[idx_row], sem)
        s2 = pltpu.async_copy(ea_v, ea_hbm.at[idx_row], sem)
        s3 = pltpu.async_copy(eas_v, eas_hbm.at[idx_row], sem)
        s1.wait()
        s2.wait()
        s3.wait()
        return 0

    lax.fori_loop(0, n_chunks, chunk_body, 0)


def kernel(param, grad, exp_avg, exp_avg_sq, index):
    M, D = param.shape
    B = index.shape[0]
    assert B % (NW * 4 * CHUNK) == 0 and D == 64

    idxr = _remap(index.astype(jnp.int32), TCB_T)
    idx2d = idxr.reshape(B // CHUNK, CHUNK)

    p2, ea2, eas2 = _t_in([param.T, exp_avg.T, exp_avg_sq.T], TCB_T)
    (g2,) = _t_in([grad.T], TCB_G)

    mpad = 2 * p2.shape[0]
    p_ref = jax.new_ref(p2.reshape(mpad, D))
    ea_ref = jax.new_ref(ea2.reshape(mpad, D))
    eas_ref = jax.new_ref(eas2.reshape(mpad, D))

    mesh = plsc.VectorSubcoreMesh(
        core_axis_name="c", subcore_axis_name="s",
        num_cores=NC, num_subcores=NS)
    n_chunks = (B // CHUNK) // NW
    _vm = lambda: pltpu.VMEM((CHUNK, D), jnp.float32)
    sc_update = pl.kernel(
        _sc_body, out_type=(), mesh=mesh,
        compiler_params=pltpu.CompilerParams(use_tc_tiling_on_sc=False),
        scratch_types=[
            pltpu.VMEM((n_chunks, CHUNK), jnp.int32),
            _vm(), _vm(), _vm(), _vm(),
            pltpu.SemaphoreType.DMA,
        ],
    )
    sc_update(p_ref, ea_ref, eas_ref, g2, idx2d)
    po, eao, easo = _t_out(
        [p_ref[...].reshape(mpad // 2, 2 * D),
         ea_ref[...].reshape(mpad // 2, 2 * D),
         eas_ref[...].reshape(mpad // 2, 2 * D)], M, TCB_T)
    return po.T, eao.T, easo.T
